# baseline (speedup 1.0000x reference)
import functools

import jax
import jax.numpy as jnp
from jax import lax
from jax.experimental import pallas as pl
from jax.experimental.pallas import tpu as pltpu
from jax.experimental.pallas import tpu_sc as plsc

_V = 100000
_D = 64
_BATCH = 4096
_HIST = 50
_B = _BATCH * _HIST
_NC = 2
_NS = 16
_NW = _NC * _NS
_BPW = _B // _NW
_BATCH_PW = _BATCH // _NW

_RSTRIDE = 6248
_RLEN = 6280
_RCHUNK = 128
_RFULL = 49
_RTAIL = _RLEN - _RFULL * _RCHUNK

_STEP_ROWS = 100
_GWIN = 104
_BPC = _STEP_ROWS // _HIST
_NSTEP = _BPW // _STEP_ROWS
_NBUF = 4

_mesh = plsc.VectorSubcoreMesh(core_axis_name="c", subcore_axis_name="s")


@functools.partial(
    pl.kernel,
    mesh=_mesh,
    out_type=jax.ShapeDtypeStruct((_BATCH, _HIST, _D), jnp.float32),
    scratch_types=[
        pltpu.HBM((_V, _D), jnp.float32),
        pltpu.VMEM((2, _RCHUNK, _D), jnp.float32),
        pltpu.VMEM((_NBUF, _GWIN, _D), jnp.float32),
    ]
    + [pltpu.VMEM((_GWIN,), jnp.int32)] * _NBUF
    + [
        pltpu.SemaphoreType.DMA,
        pltpu.SemaphoreType.DMA,
        pltpu.SemaphoreType.DMA,
        pltpu.SemaphoreType.DMA,
    ]
    + [pltpu.SemaphoreType.DMA] * (2 * _NBUF),
)
def _emb_kernel(table, idx, out, wide, stage, ring, *rest):
    idxb = rest[:_NBUF]
    rin = rest[_NBUF:_NBUF + 2]
    rout = rest[_NBUF + 2:_NBUF + 4]
    sems = rest[_NBUF + 4:]
    gsem = sems[:_NBUF]
    osem = sems[_NBUF:]
    cid = lax.axis_index("c")
    sid = lax.axis_index("s")
    wid = sid * _NC + cid
    base = wid * _BPW

    r0 = sid * _RSTRIDE

    def rp_in(k, b, n=_RCHUNK):
        return pltpu.make_async_copy(
            table.at[pl.ds(r0 + k * _RCHUNK, n)],
            stage.at[b].at[pl.ds(0, n)], rin[b])

    def rp_out(k, b, n=_RCHUNK):
        return pltpu.make_async_copy(
            stage.at[b].at[pl.ds(0, n)],
            wide.at[pl.ds(r0 + k * _RCHUNK, n)], rout[b])

    rp_in(0, 0).start()

    def rp_group(g, carry):
        for b in range(2):
            k = 2 * g + b
            rp_in(k, b).wait()

            @pl.when(k >= 1)
            def _():
                rp_out(k - 1, 1 - b).wait()

            rp_in(k + 1, 1 - b).start()
            rp_out(k, b).start()
        return carry

    lax.fori_loop(0, (_RFULL - 1) // 2, rp_group, 0)
    rp_in(_RFULL - 1, 0).wait()
    rp_out(_RFULL - 2, 1).wait()
    rp_in(_RFULL, 1, _RTAIL).start()
    rp_out(_RFULL - 1, 0).start()
    rp_in(_RFULL, 1, _RTAIL).wait()
    rp_out(_RFULL, 1, _RTAIL).start()
    rp_out(_RFULL - 1, 0).wait()
    rp_out(_RFULL, 1, _RTAIL).wait()

    plsc.subcore_barrier()

    def gpad(s):
        return 4 * lax.rem(s, 2)

    def g_start(s, b):
        off = pl.multiple_of(base + s * _STEP_ROWS - gpad(s), 8)
        pltpu.sync_copy(idx.at[pl.ds(off, _GWIN)], idxb[b])
        pltpu.make_async_copy(wide.at[idxb[b]], ring.at[b], gsem[b]).start()

    def g_copy(s, b):
        return pltpu.make_async_copy(wide.at[idxb[b]], ring.at[b], gsem[b])

    def o_copy(s, b, j):
        return pltpu.make_async_copy(
            ring.at[b].at[pl.ds(gpad(s) + j * _HIST, _HIST)],
            out.at[wid * _BATCH_PW + s * _BPC + j],
            osem[b])

    for b in range(_NBUF):
        g_start(b, b)

    def step(g, carry):
        for b in range(_NBUF):
            s = g * _NBUF + b
            g_copy(s, b).wait()
            for j in range(_BPC):
                o_copy(s, b, j).start()
            b1 = (b - 1) % _NBUF
            s_prev = s - 1
            s_next = s + _NBUF - 1

            @pl.when(jnp.logical_and(s_prev >= 0, s_next < _NSTEP))
            def _():
                for j in range(_BPC):
                    o_copy(s_prev, b1, j).wait()
                g_start(s_next, b1)

        return carry

    lax.fori_loop(0, _NSTEP // _NBUF, step, 0)

    for k in range(_NBUF):
        s = _NSTEP - _NBUF + k
        for j in range(_BPC):
            o_copy(s, s % _NBUF, j).wait()


def kernel(input_ids, embedding):
    idx = input_ids.reshape(-1).astype(jnp.int32)
    return _emb_kernel(embedding, idx)


# speedup vs baseline: 1.1484x; 1.1484x over previous
import functools

import jax
import jax.numpy as jnp
from jax import lax
from jax.experimental import pallas as pl
from jax.experimental.pallas import tpu as pltpu
from jax.experimental.pallas import tpu_sc as plsc

_D = 64
_B = 4096 * 50
_NC = 2
_NS = 16
_NW = _NC * _NS
_BPW = _B // _NW
_CHUNK = 256
_NSTEP = _BPW // _CHUNK
_NBUF = 5
_NGRP = _NSTEP // _NBUF

_mesh = plsc.VectorSubcoreMesh(core_axis_name="c", subcore_axis_name="s")


@functools.partial(
    pl.kernel,
    mesh=_mesh,
    out_type=jax.ShapeDtypeStruct((_B, _D), jnp.float32),
    scratch_types=[
        pltpu.VMEM((_BPW,), jnp.int32),
        pltpu.VMEM((_NBUF, _CHUNK, _D), jnp.float32),
    ]
    + [pltpu.SemaphoreType.DMA] * (2 * _NBUF),
    compiler_params=pltpu.CompilerParams(use_tc_tiling_on_sc=False),
)
def _gather_kernel(table, idx, out, idx_v, rows_v, *sems):
    gsem = sems[:_NBUF]
    osem = sems[_NBUF:]
    wid = lax.axis_index("s") * _NC + lax.axis_index("c")
    base = wid * _BPW
    pltpu.sync_copy(idx.at[pl.ds(base, _BPW)], idx_v)

    def g_copy(s, b):
        return pltpu.make_async_copy(
            table.at[idx_v.at[pl.ds(s * _CHUNK, _CHUNK)]], rows_v.at[b], gsem[b])

    def o_copy(s, b):
        return pltpu.make_async_copy(
            rows_v.at[b], out.at[pl.ds(base + s * _CHUNK, _CHUNK)], osem[b])

    for b in range(_NBUF):
        g_copy(b, b).start()

    def group(g, carry):
        for b in range(_NBUF):
            s = g * _NBUF + b
            g_copy(s, b).wait()
            o_copy(s, b).start()
            b1 = (b - 1) % _NBUF
            s_prev = s - 1
            s_next = s + _NBUF - 1

            @pl.when(jnp.logical_and(s_prev >= 0, s_next < _NSTEP))
            def _():
                o_copy(s_prev, b1).wait()
                g_copy(s_next, b1).start()

        return carry

    lax.fori_loop(0, _NGRP, group, 0)

    for k in range(_NBUF):
        s = _NSTEP - _NBUF + k
        o_copy(s, s % _NBUF).wait()


def kernel(input_ids, embedding):
    idx = input_ids.reshape(-1).astype(jnp.int32)
    out = _gather_kernel(embedding, idx)
    return out.reshape(input_ids.shape + (_D,))


# speedup vs baseline: 1.1537x; 1.0047x over previous
import functools

import jax
import jax.numpy as jnp
from jax import lax
from jax.experimental import pallas as pl
from jax.experimental.pallas import tpu as pltpu
from jax.experimental.pallas import tpu_sc as plsc

_D = 64
_B = 4096 * 50
_NC = 2
_NS = 16
_NW = _NC * _NS
_BPW = _B // _NW
_CHUNK = 128
_NSTEP = _BPW // _CHUNK
_NBUF = 5
_NGRP = _NSTEP // _NBUF

_mesh = plsc.VectorSubcoreMesh(core_axis_name="c", subcore_axis_name="s")


@functools.partial(
    pl.kernel,
    mesh=_mesh,
    out_type=jax.ShapeDtypeStruct((_B, _D), jnp.float32),
    scratch_types=[
        pltpu.VMEM((_BPW,), jnp.int32),
        pltpu.VMEM((_NBUF, _CHUNK, _D), jnp.float32),
    ]
    + [pltpu.SemaphoreType.DMA] * (2 * _NBUF),
    compiler_params=pltpu.CompilerParams(use_tc_tiling_on_sc=False),
)
def _gather_kernel(table, idx, out, idx_v, rows_v, *sems):
    gsem = sems[:_NBUF]
    osem = sems[_NBUF:]
    wid = lax.axis_index("s") * _NC + lax.axis_index("c")
    base = wid * _BPW
    pltpu.sync_copy(idx.at[pl.ds(base, _BPW)], idx_v)

    def g_copy(s, b):
        return pltpu.make_async_copy(
            table.at[idx_v.at[pl.ds(s * _CHUNK, _CHUNK)]], rows_v.at[b], gsem[b])

    def o_copy(s, b):
        return pltpu.make_async_copy(
            rows_v.at[b], out.at[pl.ds(base + s * _CHUNK, _CHUNK)], osem[b])

    for b in range(_NBUF):
        g_copy(b, b).start()

    def group(g, carry):
        for b in range(_NBUF):
            s = g * _NBUF + b
            g_copy(s, b).wait()
            o_copy(s, b).start()
            b1 = (b - 1) % _NBUF
            s_prev = s - 1
            s_next = s + _NBUF - 1

            @pl.when(jnp.logical_and(s_prev >= 0, s_next < _NSTEP))
            def _():
                o_copy(s_prev, b1).wait()
                g_copy(s_next, b1).start()

        return carry

    lax.fori_loop(0, _NGRP, group, 0)

    for k in range(_NBUF):
        s = _NSTEP - _NBUF + k
        o_copy(s, s % _NBUF).wait()


def kernel(input_ids, embedding):
    idx = input_ids.reshape(-1).astype(jnp.int32)
    out = _gather_kernel(embedding, idx)
    return out.reshape(input_ids.shape + (_D,))
